# Initial kernel scaffold; baseline (speedup 1.0000x reference)
import jax
import jax.numpy as jnp
from jax.experimental import pallas as pl


def kernel(edge_index, edge_dist, node_types, graph_ids, W_emb1, W_emb2, duplet_src, duplet_dst, Wf1, Wf2, Wl, W_fc, b_fc):
    raise NotImplementedError("write your pallas kernel here")


# speedup vs baseline: 2.1415x; 2.1415x over previous
import functools

import jax
import jax.numpy as jnp
from jax import lax
from jax.experimental import pallas as pl
from jax.experimental.pallas import tpu as pltpu
from jax.experimental.pallas import tpu_sc as plsc

N = 10000
E = 320000
H = 128
C = 10
G = 64

NC = 2
NS = 16
NW = NC * NS
EW = E // NW
B = 80
NB = EW // B
NPAD = 10112
ROWS_PER_TILE = NPAD // NS

@functools.cache
def _sc_mesh():
    return plsc.VectorSubcoreMesh(
        core_axis_name="c", subcore_axis_name="s", num_cores=NC, num_subcores=NS
    )


BE = 3200


def _filters_body(d_ref, we1_ref, we2_ref, wf1_ref, wf2_ref, m_ref, f_ref):
    d = d_ref[...]
    inv = jnp.float32(1.0) / jnp.float32(H - 1)
    centers = lax.broadcasted_iota(jnp.int32, (BE, H), 1).astype(jnp.float32) * inv
    gamma = jnp.float32(1.0) / (inv * inv)
    diff = d - centers
    rbf = jnp.exp(-gamma * diff * diff)
    cut = jnp.where(d < 1.0, 0.5 * (jnp.cos(jnp.pi * d) + 1.0), 0.0).astype(jnp.float32)
    bf = rbf * cut
    dot = functools.partial(jnp.dot, preferred_element_type=jnp.float32)
    m_ref[...] = dot(jnp.maximum(dot(bf, we1_ref[...]), 0.0), we2_ref[...])
    for i in range(3):
        h1 = jnp.maximum(dot(bf, wf1_ref[i]), 0.0)
        f_ref[i] = dot(h1, wf2_ref[i]) * cut


def _filters(d2, we1, we2, wf1, wf2):
    grid = E // BE
    return pl.pallas_call(
        _filters_body,
        grid=(grid,),
        in_specs=[
            pl.BlockSpec((BE, 1), lambda i: (i, 0)),
            pl.BlockSpec((H, H), lambda i: (0, 0)),
            pl.BlockSpec((H, H), lambda i: (0, 0)),
            pl.BlockSpec((3, H, H), lambda i: (0, 0, 0)),
            pl.BlockSpec((3, H, H), lambda i: (0, 0, 0)),
        ],
        out_specs=[
            pl.BlockSpec((BE, H), lambda i: (i, 0)),
            pl.BlockSpec((3, BE, H), lambda i: (0, i, 0)),
        ],
        out_shape=[
            jax.ShapeDtypeStruct((E, H), jnp.float32),
            jax.ShapeDtypeStruct((3, E, H), jnp.float32),
        ],
    )(d2, we1, we2, wf1, wf2)


def _sc_zero_and_ids(acc, zeros_hbm):
    c = lax.axis_index("c")
    s = lax.axis_index("s")
    w = c * NS + s
    pltpu.sync_copy(zeros_hbm, acc.at[pl.ds(s * ROWS_PER_TILE, ROWS_PER_TILE)])
    plsc.subcore_barrier()
    return c, s, w


def _sc_writeout(acc, out_hbm, c, s):
    plsc.subcore_barrier()
    pltpu.sync_copy(
        acc.at[pl.ds(s * ROWS_PER_TILE, ROWS_PER_TILE)],
        out_hbm.at[c, pl.ds(s * ROWS_PER_TILE, ROWS_PER_TILE)],
    )


def _sc_embed_body(m_hbm, idx4_hbm, zeros_hbm, out_hbm, idxb, rows, acc):
    c, s, w = _sc_zero_and_ids(acc, zeros_hbm)

    def blk(b, carry):
        pltpu.sync_copy(idx4_hbm.at[w, b], idxb)
        pltpu.sync_copy(m_hbm.at[pl.ds(w * EW + b * B, B)], rows)
        pltpu.sync_copy(rows, acc.at[idxb.at[1]], add=True)
        return carry

    lax.fori_loop(0, NB, blk, 0)
    _sc_writeout(acc, out_hbm, c, s)


def _sc_embed(m, idx4, zeros):
    return pl.kernel(
        _sc_embed_body,
        out_type=jax.ShapeDtypeStruct((NC, NPAD, H), jnp.float32),
        mesh=_sc_mesh(),
        scratch_types=[
            pltpu.VMEM((2, B), jnp.int32),
            pltpu.VMEM((B, H), jnp.float32),
            pltpu.VMEM_SHARED((NPAD, H), jnp.float32),
        ],
    )(m, idx4, zeros)


def _make_sc_conv():
    def body(x_hbm, f_hbm, idx4_hbm, zeros_hbm, out_hbm,
             idxb, xrows, frows, acc, sem):
        c, s, w = _sc_zero_and_ids(acc, zeros_hbm)

        def blk(b, carry):
            pltpu.sync_copy(idx4_hbm.at[w, b], idxb)
            pltpu.async_copy(x_hbm.at[idxb.at[0]], xrows, sem).wait()
            pltpu.sync_copy(f_hbm.at[pl.ds(w * EW + b * B, B)], frows)

            def mul_row(r, c2):
                for j in range(H // 16):
                    sl = pl.ds(j * 16, 16)
                    xrows[r, sl] = xrows[r, sl] * frows[r, sl]
                return c2

            lax.fori_loop(0, B, mul_row, 0)
            pltpu.sync_copy(xrows, acc.at[idxb.at[1]], add=True)
            return carry

        lax.fori_loop(0, NB, blk, 0)
        _sc_writeout(acc, out_hbm, c, s)

    return pl.kernel(
        body,
        out_type=jax.ShapeDtypeStruct((NC, NPAD, H), jnp.float32),
        mesh=_sc_mesh(),
        scratch_types=[
            pltpu.VMEM((2, B), jnp.int32),
            pltpu.VMEM((B, H), jnp.float32),
            pltpu.VMEM((B, H), jnp.float32),
            pltpu.VMEM_SHARED((NPAD, H), jnp.float32),
            pltpu.SemaphoreType.DMA,
        ],
    )


def _combine_body(p_ref, o_ref):
    o_ref[...] = p_ref[0] + p_ref[1]


def _combine(parts):
    return pl.pallas_call(
        _combine_body,
        out_shape=jax.ShapeDtypeStruct((NPAD, H), jnp.float32),
    )(parts)


def _update_body(p_ref, w_ref, o_ref):
    agg = p_ref[0] + p_ref[1]
    o_ref[...] = jnp.maximum(
        jnp.dot(agg, w_ref[...], preferred_element_type=jnp.float32), 0.0
    )


def _update(parts, w):
    return pl.pallas_call(
        _update_body,
        out_shape=jax.ShapeDtypeStruct((NPAD, H), jnp.float32),
    )(parts, w)


def _head_body(p_ref, wl_ref, gid_ref, wfc_ref, bfc_ref, o_ref):
    agg = p_ref[0] + p_ref[1]
    x = jnp.maximum(jnp.dot(agg, wl_ref[...], preferred_element_type=jnp.float32), 0.0)
    gid = gid_ref[...]
    onehot = (gid == lax.broadcasted_iota(jnp.int32, (NPAD, G), 1)).astype(jnp.float32)
    sums = lax.dot_general(
        onehot, x, (((0,), (0,)), ((), ())), preferred_element_type=jnp.float32
    )
    cnt = jnp.sum(onehot, axis=0)[:, None]
    xg = sums / jnp.maximum(cnt, 1.0)
    logits = jnp.dot(xg, wfc_ref[...], preferred_element_type=jnp.float32) + bfc_ref[...]
    mx = jnp.max(logits, axis=1, keepdims=True)
    z = logits - mx
    lse = jnp.log(jnp.sum(jnp.exp(z), axis=1, keepdims=True))
    o_ref[...] = z - lse


def _head(parts, wl, gid2, wfc, bfc2):
    return pl.pallas_call(
        _head_body,
        out_shape=jax.ShapeDtypeStruct((G, C), jnp.float32),
    )(parts, wl, gid2, wfc, bfc2)


def kernel(edge_index, edge_dist, node_types, graph_ids, W_emb1, W_emb2,
           duplet_src, duplet_dst, Wf1, Wf2, Wl, W_fc, b_fc):
    del node_types, duplet_src, duplet_dst
    src3 = edge_index[0].astype(jnp.int32).reshape(NW, NB, 1, B)
    dst3 = edge_index[1].astype(jnp.int32).reshape(NW, NB, 1, B)
    idx4 = jnp.concatenate([src3, dst3], axis=2)
    d2 = edge_dist.astype(jnp.float32).reshape(E, 1)
    zeros = jnp.zeros((ROWS_PER_TILE, H), jnp.float32)
    gid2 = jnp.concatenate(
        [graph_ids.astype(jnp.int32), jnp.full((NPAD - N,), G, jnp.int32)]
    ).reshape(NPAD, 1)
    bfc2 = b_fc.reshape(1, C)

    m, filt = _filters(d2, W_emb1, W_emb2, Wf1, Wf2)
    parts = _sc_embed(m, idx4, zeros)
    x = _combine(parts)
    sc_conv = _make_sc_conv()
    for i in range(2):
        parts = sc_conv(x, filt[i], idx4, zeros)
        x = _update(parts, Wl[i])
    parts = sc_conv(x, filt[2], idx4, zeros)
    return _head(parts, Wl[2], gid2, W_fc, bfc2)
